# baseline (speedup 1.0000x reference)
import functools

import jax
import jax.numpy as jnp
from jax.experimental import pallas as pl
from jax.experimental.pallas import tpu as pltpu

_TILE = 2048
_CHUNKS = (12288, 20480, 20480, 12288)


def _partials_kernel(u_ref, p_ref, n_ref, out_ref):
    u = u_ref[...]
    p = p_ref[...]
    n = n_ref[...]

    a = jnp.sum(u * p, axis=1, keepdims=True)
    b = jnp.sum(u * n, axis=1, keepdims=True)
    sq = jnp.sum(u * u + p * p + n * n)

    sp = 1.0 / (1.0 + jnp.exp(-a))
    sn = 1.0 / (1.0 + jnp.exp(-b))
    wmf = jnp.sum(2.0 * (sp - 1.0) ** 2 + sn * sn)

    lane = jax.lax.broadcasted_iota(jnp.int32, (1, 8, 128), 2)
    sub = jax.lax.broadcasted_iota(jnp.int32, (1, 8, 128), 1)
    out_ref[...] = jnp.where((lane == 0) & (sub == 0), sq, 0.0) + \
                   jnp.where((lane == 1) & (sub == 0), wmf, 0.0)


def _chunk_call(bc):
    tpc = bc // (2 * _TILE)
    u_spec = pl.BlockSpec((_TILE, 128), lambda c, t: (c * tpc + t, 0))
    p_spec = pl.BlockSpec((_TILE, 128), lambda c, t: (c * tpc + t, 0))
    n_spec = pl.BlockSpec((_TILE, 128),
                          lambda c, t, _n=2 * tpc: (_n + c * tpc + t, 0))
    return pl.pallas_call(
        _partials_kernel,
        out_shape=jax.ShapeDtypeStruct((2 * tpc, 8, 128), jnp.float32),
        grid=(2, tpc),
        in_specs=[u_spec, p_spec, n_spec],
        out_specs=pl.BlockSpec((1, 8, 128), lambda c, t: (c * tpc + t, 0, 0)),
        compiler_params=pltpu.CompilerParams(
            dimension_semantics=("parallel", "arbitrary")),
    )


def kernel(user_embedding, item_embedding, users, positive_items,
           negative_items, weight_decay):
    B = users.shape[0]
    assert sum(_CHUNKS) == B and all(c % (2 * _TILE) == 0 for c in _CHUNKS)

    partial_sums = []
    off = 0
    for bc in _CHUNKS:
        us = jax.lax.dynamic_slice_in_dim(users, off, bc)
        ps = jax.lax.dynamic_slice_in_dim(positive_items, off, bc)
        ns = jax.lax.dynamic_slice_in_dim(negative_items, off, bc)
        off += bc
        u = user_embedding[us]
        pn = item_embedding[jnp.concatenate([ps, ns])]
        part = _chunk_call(bc)(u, pn, pn)
        partial_sums.append(jnp.sum(part[:, 0, 0:2], axis=0))

    totals = jnp.sum(jnp.stack(partial_sums), axis=0)
    sq_total, wmf_total = totals[0], totals[1]
    return wmf_total / (2.0 * B) + weight_decay * 0.5 * sq_total / B


# speedup vs baseline: 1.0735x; 1.0735x over previous
import functools

import jax
import jax.numpy as jnp
from jax.experimental import pallas as pl
from jax.experimental.pallas import tpu as pltpu

_TILE = 2048
_CHUNKS = (24576, 24576, 16384)


def _partials_kernel(u_ref, p_ref, n_ref, out_ref):
    u = u_ref[...]
    p = p_ref[...]
    n = n_ref[...]

    a = jnp.sum(u * p, axis=1, keepdims=True)
    b = jnp.sum(u * n, axis=1, keepdims=True)
    sq = jnp.sum(u * u + p * p + n * n)

    sp = 1.0 / (1.0 + jnp.exp(-a))
    sn = 1.0 / (1.0 + jnp.exp(-b))
    wmf = jnp.sum(2.0 * (sp - 1.0) ** 2 + sn * sn)

    lane = jax.lax.broadcasted_iota(jnp.int32, (1, 8, 128), 2)
    sub = jax.lax.broadcasted_iota(jnp.int32, (1, 8, 128), 1)
    out_ref[...] = jnp.where((lane == 0) & (sub == 0), sq, 0.0) + \
                   jnp.where((lane == 1) & (sub == 0), wmf, 0.0)


def _chunk_call(bc):
    tpc = bc // (2 * _TILE)
    u_spec = pl.BlockSpec((_TILE, 128), lambda c, t: (c * tpc + t, 0))
    p_spec = pl.BlockSpec((_TILE, 128), lambda c, t: (c * tpc + t, 0))
    n_spec = pl.BlockSpec((_TILE, 128),
                          lambda c, t, _n=2 * tpc: (_n + c * tpc + t, 0))
    return pl.pallas_call(
        _partials_kernel,
        out_shape=jax.ShapeDtypeStruct((2 * tpc, 8, 128), jnp.float32),
        grid=(2, tpc),
        in_specs=[u_spec, p_spec, n_spec],
        out_specs=pl.BlockSpec((1, 8, 128), lambda c, t: (c * tpc + t, 0, 0)),
        compiler_params=pltpu.CompilerParams(
            dimension_semantics=("parallel", "arbitrary")),
    )


def kernel(user_embedding, item_embedding, users, positive_items,
           negative_items, weight_decay):
    B = users.shape[0]
    assert sum(_CHUNKS) == B and all(c % (2 * _TILE) == 0 for c in _CHUNKS)

    partial_sums = []
    off = 0
    for bc in _CHUNKS:
        us = jax.lax.dynamic_slice_in_dim(users, off, bc)
        ps = jax.lax.dynamic_slice_in_dim(positive_items, off, bc)
        ns = jax.lax.dynamic_slice_in_dim(negative_items, off, bc)
        off += bc
        u = user_embedding[us]
        pn = item_embedding[jnp.concatenate([ps, ns])]
        part = _chunk_call(bc)(u, pn, pn)
        partial_sums.append(jnp.sum(part[:, 0, 0:2], axis=0))

    totals = jnp.sum(jnp.stack(partial_sums), axis=0)
    sq_total, wmf_total = totals[0], totals[1]
    return wmf_total / (2.0 * B) + weight_decay * 0.5 * sq_total / B


# speedup vs baseline: 1.0748x; 1.0012x over previous
import jax
import jax.numpy as jnp
from jax.experimental import pallas as pl
from jax.experimental.pallas import tpu as pltpu

_TILE = 2048
_CHUNKS = (24576, 24576, 16384)


def _partials_kernel(u_ref, p_ref, n_ref, out_ref):
    u = u_ref[...]
    p = p_ref[...]
    n = n_ref[...]

    a = jnp.sum(u * p, axis=1, keepdims=True)
    b = jnp.sum(u * n, axis=1, keepdims=True)
    sq = jnp.sum(u * u + p * p + n * n)

    sp = 1.0 / (1.0 + jnp.exp(-a))
    sn = 1.0 / (1.0 + jnp.exp(-b))
    wmf = jnp.sum(2.0 * (sp - 1.0) ** 2 + sn * sn)

    lane = jax.lax.broadcasted_iota(jnp.int32, (1, 8, 128), 2)
    sub = jax.lax.broadcasted_iota(jnp.int32, (1, 8, 128), 1)
    out_ref[...] = jnp.where((lane == 0) & (sub == 0), sq, 0.0) + \
                   jnp.where((lane == 1) & (sub == 0), wmf, 0.0)


def _chunk_call(bc):
    tpc = bc // (2 * _TILE)
    u_spec = pl.BlockSpec((_TILE, 128), lambda c, t: (c * tpc + t, 0))
    p_spec = pl.BlockSpec((_TILE, 128), lambda c, t: (c * tpc + t, 0))
    n_spec = pl.BlockSpec((_TILE, 128),
                          lambda c, t, _n=2 * tpc: (_n + c * tpc + t, 0))
    return pl.pallas_call(
        _partials_kernel,
        out_shape=jax.ShapeDtypeStruct((2 * tpc, 8, 128), jnp.float32),
        grid=(2, tpc),
        in_specs=[u_spec, p_spec, n_spec],
        out_specs=pl.BlockSpec((1, 8, 128), lambda c, t: (c * tpc + t, 0, 0)),
        compiler_params=pltpu.CompilerParams(
            dimension_semantics=("parallel", "arbitrary")),
    )


def kernel(user_embedding, item_embedding, users, positive_items,
           negative_items, weight_decay):
    B = users.shape[0]
    assert sum(_CHUNKS) == B and all(c % (2 * _TILE) == 0 for c in _CHUNKS)

    partial_sums = []
    off = 0
    for bc in _CHUNKS:
        us = jax.lax.dynamic_slice_in_dim(users, off, bc)
        ps = jax.lax.dynamic_slice_in_dim(positive_items, off, bc)
        ns = jax.lax.dynamic_slice_in_dim(negative_items, off, bc)
        off += bc
        u = user_embedding[us]
        pn = item_embedding[jnp.concatenate([ps, ns])]
        part = _chunk_call(bc)(u, pn, pn)
        partial_sums.append(jnp.sum(part[:, 0, 0:2], axis=0))

    totals = jnp.sum(jnp.stack(partial_sums), axis=0)
    sq_total, wmf_total = totals[0], totals[1]
    return wmf_total / (2.0 * B) + weight_decay * 0.5 * sq_total / B
